# Initial kernel scaffold; baseline (speedup 1.0000x reference)
import jax
import jax.numpy as jnp
from jax.experimental import pallas as pl


def kernel(feats, w_conv, W1, b1, W2, b2, W3, b3):
    raise NotImplementedError("write your pallas kernel here")


# speedup vs baseline: 1.1424x; 1.1424x over previous
import jax
import jax.numpy as jnp
from jax import lax
from jax.experimental import pallas as pl
from jax.experimental.pallas import tpu as pltpu

_B, _N, _F, _R = 8, 8192, 2048, 5
_BLK = 1024
_NSTEPS = (_B * _N) // _BLK
_PER_BATCH = _N // _BLK


def _sigmoid(x):
    return 1.0 / (1.0 + jnp.exp(-x))


def _body(feats_ref, w_ref, W1_ref, b1_ref, W2_ref, b2_ref, W3_ref, b3_ref,
          logits_ref, probs_ref, scores_ref):
    j = pl.program_id(0)
    part = jnp.dot(feats_ref[...], w_ref[...],
                   preferred_element_type=jnp.float32)
    row = part.reshape(1, _BLK)
    b = j // _PER_BATCH
    n0 = (j % _PER_BATCH) * _BLK
    scores_ref[pl.ds(b, 1), pl.ds(n0, _BLK)] = row

    @pl.when(j == _NSTEPS - 1)
    def _finish():
        s = scores_ref[...]
        iota = lax.broadcasted_iota(jnp.int32, (_B, _N), 1)
        big = jnp.int32(_N)
        picks = []
        v = s
        for _ in range(_R):
            m = jnp.min(v, axis=1, keepdims=True)
            picks.append(m)
            idx = jnp.min(jnp.where(v == m, iota, big), axis=1, keepdims=True)
            v = jnp.where(iota == idx, jnp.inf, v)
        tops = []
        v = s
        for _ in range(_R):
            m = jnp.max(v, axis=1, keepdims=True)
            tops.append(m)
            idx = jnp.min(jnp.where(v == m, iota, big), axis=1, keepdims=True)
            v = jnp.where(iota == idx, -jnp.inf, v)
        x = jnp.concatenate(picks + tops[::-1], axis=1)
        h = _sigmoid(jnp.dot(x, W1_ref[...],
                             preferred_element_type=jnp.float32) + b1_ref[...])
        h = _sigmoid(jnp.dot(h, W2_ref[...],
                             preferred_element_type=jnp.float32) + b2_ref[...])
        lg = jnp.dot(h, W3_ref[...],
                     preferred_element_type=jnp.float32) + b3_ref[...]
        logits_ref[...] = lg
        probs_ref[...] = _sigmoid(lg)


def kernel(feats, w_conv, W1, b1, W2, b2, W3, b3):
    feats2d = feats.reshape(_B * _N, _F)
    w2d = w_conv.reshape(_F, 1)
    b1r = b1.reshape(1, -1)
    b2r = b2.reshape(1, -1)
    b3r = b3.reshape(1, -1)
    logits, probs = pl.pallas_call(
        _body,
        grid=(_NSTEPS,),
        in_specs=[
            pl.BlockSpec((_BLK, _F), lambda j: (j, 0)),
            pl.BlockSpec((_F, 1), lambda j: (0, 0)),
            pl.BlockSpec(W1.shape, lambda j: (0, 0)),
            pl.BlockSpec((1, b1.shape[0]), lambda j: (0, 0)),
            pl.BlockSpec(W2.shape, lambda j: (0, 0)),
            pl.BlockSpec((1, b2.shape[0]), lambda j: (0, 0)),
            pl.BlockSpec(W3.shape, lambda j: (0, 0)),
            pl.BlockSpec((1, 1), lambda j: (0, 0)),
        ],
        out_specs=[
            pl.BlockSpec((_B, 1), lambda j: (0, 0)),
            pl.BlockSpec((_B, 1), lambda j: (0, 0)),
        ],
        out_shape=[
            jax.ShapeDtypeStruct((_B, 1), jnp.float32),
            jax.ShapeDtypeStruct((_B, 1), jnp.float32),
        ],
        scratch_shapes=[pltpu.VMEM((_B, _N), jnp.float32)],
    )(feats2d, w2d, W1, b1r, W2, b2r, W3, b3r)
    return logits, probs


# speedup vs baseline: 1.2363x; 1.0822x over previous
import jax
import jax.numpy as jnp
from jax import lax
from jax.experimental import pallas as pl
from jax.experimental.pallas import tpu as pltpu

_B, _N, _F, _R = 8, 8192, 2048, 5
_BLK = 1024
_NSTEPS = (_B * _N) // _BLK
_PER_BATCH = _N // _BLK


def _sigmoid(x):
    return 1.0 / (1.0 + jnp.exp(-x))


def _body(feats_ref, w_ref, W1_ref, b1_ref, W2_ref, b2_ref, W3_ref, b3_ref,
          logits_ref, probs_ref, scores_ref):
    j = pl.program_id(0)
    row = lax.dot_general(w_ref[...], feats_ref[...],
                          (((0,), (1,)), ((), ())),
                          preferred_element_type=jnp.float32)
    scores_ref[pl.ds(j, 1), :] = row

    @pl.when(j == _NSTEPS - 1)
    def _finish():
        s = scores_ref[...].reshape(_B, _N)
        iota = lax.broadcasted_iota(jnp.int32, (_B, _N), 1)
        big = jnp.int32(_N)
        picks = []
        v = s
        for _ in range(_R):
            m = jnp.min(v, axis=1, keepdims=True)
            picks.append(m)
            idx = jnp.min(jnp.where(v == m, iota, big), axis=1, keepdims=True)
            v = jnp.where(iota == idx, jnp.inf, v)
        tops = []
        v = s
        for _ in range(_R):
            m = jnp.max(v, axis=1, keepdims=True)
            tops.append(m)
            idx = jnp.min(jnp.where(v == m, iota, big), axis=1, keepdims=True)
            v = jnp.where(iota == idx, -jnp.inf, v)
        x = jnp.concatenate(picks + tops[::-1], axis=1)
        h = _sigmoid(jnp.dot(x, W1_ref[...],
                             preferred_element_type=jnp.float32) + b1_ref[...])
        h = _sigmoid(jnp.dot(h, W2_ref[...],
                             preferred_element_type=jnp.float32) + b2_ref[...])
        lg = jnp.dot(h, W3_ref[...],
                     preferred_element_type=jnp.float32) + b3_ref[...]
        logits_ref[...] = lg
        probs_ref[...] = _sigmoid(lg)


def kernel(feats, w_conv, W1, b1, W2, b2, W3, b3):
    feats2d = feats.reshape(_B * _N, _F)
    w2d = w_conv.reshape(_F, 1)
    b1r = b1.reshape(1, -1)
    b2r = b2.reshape(1, -1)
    b3r = b3.reshape(1, -1)
    logits, probs = pl.pallas_call(
        _body,
        grid=(_NSTEPS,),
        in_specs=[
            pl.BlockSpec((_BLK, _F), lambda j: (j, 0)),
            pl.BlockSpec((_F, 1), lambda j: (0, 0)),
            pl.BlockSpec(W1.shape, lambda j: (0, 0)),
            pl.BlockSpec((1, b1.shape[0]), lambda j: (0, 0)),
            pl.BlockSpec(W2.shape, lambda j: (0, 0)),
            pl.BlockSpec((1, b2.shape[0]), lambda j: (0, 0)),
            pl.BlockSpec(W3.shape, lambda j: (0, 0)),
            pl.BlockSpec((1, 1), lambda j: (0, 0)),
        ],
        out_specs=[
            pl.BlockSpec((_B, 1), lambda j: (0, 0)),
            pl.BlockSpec((_B, 1), lambda j: (0, 0)),
        ],
        out_shape=[
            jax.ShapeDtypeStruct((_B, 1), jnp.float32),
            jax.ShapeDtypeStruct((_B, 1), jnp.float32),
        ],
        scratch_shapes=[pltpu.VMEM((_NSTEPS, _BLK), jnp.float32)],
    )(feats2d, w2d, W1, b1r, W2, b2r, W3, b3r)
    return logits, probs
